# baseline (speedup 1.0000x reference)
import jax
import jax.numpy as jnp
from jax import lax
from jax.experimental import pallas as pl
from jax.experimental.pallas import tpu as pltpu
from jax.experimental.pallas import tpu_sc as plsc

D = 64
L = 16
NJ = D // L
NEG = 20
R = NEG + 1
BC = 32
DU = 4


def _sc_body(cidx, ocidx, wc, wo, dots_out,
             idx_c, idx_oc, c_all, oc_rows, dots_v,
             sem_c, sem_g, sem_i, sem_o):
    info = plsc.get_sparse_core_info()
    nw = info.num_cores * info.num_subcores
    B = cidx.shape[0]
    b_per_w = B // nw
    chunks = b_per_w // BC

    wid = lax.axis_index("s") * info.num_cores + lax.axis_index("c")
    w0 = pl.multiple_of(wid * b_per_w, b_per_w)

    pltpu.sync_copy(cidx.at[pl.ds(w0, b_per_w)], idx_c)
    pltpu.async_copy(wc.at[idx_c], c_all, sem_c).wait()

    def idx_cp(ch, p):
        return pltpu.make_async_copy(
            ocidx.at[pl.ds((w0 + ch * BC) * R, BC * R)],
            idx_oc.at[p], sem_i)

    def rows_cp(p):
        return pltpu.make_async_copy(
            wo.at[idx_oc.at[p]], oc_rows.at[p], sem_g.at[p])

    def dots_cp(ch, p):
        return pltpu.make_async_copy(
            dots_v.at[p],
            dots_out.at[pl.ds((w0 + ch * BC) * R, BC * R)], sem_o)

    def compute(ch, p):
        lanes = lax.iota(jnp.int32, L)
        for bb in range(BC // L):
            b_ids = ch * BC + bb * L + lanes
            ids_r = (bb * L + lanes) * R
            for (k0, k1) in ((0, 11), (11, R)):
                nk = k1 - k0
                row_ids = [ids_r + (k0 + t) for t in range(nk)]

                def dbody(d0, accs, row_ids=row_ids, b_ids=b_ids, p=p):
                    for u in range(DU):
                        d = d0 * DU + u
                        dcol = jnp.full((L,), d, jnp.int32)
                        cvec = plsc.load_gather(c_all, [b_ids, dcol])
                        accs = tuple(
                            acc + cvec * plsc.load_gather(
                                oc_rows.at[p], [row_ids[t], dcol])
                            for t, acc in enumerate(accs))
                    return accs

                accs = lax.fori_loop(
                    0, D // DU, dbody,
                    tuple(jnp.zeros((L,), jnp.float32) for _ in range(nk)))
                for t in range(nk):
                    dots_v[p, pl.ds((k0 + t) * BC + bb * L, L)] = accs[t]

    idx_cp(0, 0).start()
    idx_cp(0, 0).wait()
    rows_cp(0).start()
    idx_cp(1, 1).start()

    def chbody(ch, carry):
        p = ch & 1

        @pl.when(ch + 1 < chunks)
        def _():
            idx_cp(ch + 1, 1 - p).wait()
            rows_cp(1 - p).start()

        @pl.when(ch + 2 < chunks)
        def _():
            idx_cp(ch + 2, p).start()

        rows_cp(p).wait()

        @pl.when(ch >= 1)
        def _():
            dots_cp(ch - 1, 1 - p).wait()

        compute(ch, p)
        dots_cp(ch, p).start()
        return carry

    lax.fori_loop(0, chunks, chbody, 0)
    dots_cp(chunks - 1, (chunks - 1) & 1).wait()


def _sc_dots(center_word, oc_idx, W_center, W_outside):
    B = center_word.shape[0]
    info = plsc.get_sparse_core_info()
    nw = info.num_cores * info.num_subcores
    b_per_w = B // nw
    mesh = plsc.VectorSubcoreMesh(core_axis_name="c", subcore_axis_name="s")
    f = pl.kernel(
        _sc_body, mesh=mesh,
        compiler_params=pltpu.CompilerParams(
            needs_layout_passes=False, use_tc_tiling_on_sc=False),
        out_type=jax.ShapeDtypeStruct((B * R,), jnp.float32),
        scratch_types=[
            pltpu.VMEM((b_per_w,), jnp.int32),
            pltpu.VMEM((2, BC * R), jnp.int32),
            pltpu.VMEM((b_per_w, D), jnp.float32),
            pltpu.VMEM((2, BC * R, D), jnp.float32),
            pltpu.VMEM((2, BC * R), jnp.float32),
            pltpu.SemaphoreType.DMA,
            pltpu.SemaphoreType.DMA((2,)),
            pltpu.SemaphoreType.DMA,
            pltpu.SemaphoreType.DMA,
        ],
    )
    return f(center_word, oc_idx, W_center, W_outside)


def _loss_body(dots_ref, out_ref):
    tot = jnp.sum(jax.nn.log_sigmoid(dots_ref[...]))
    out_ref[0, 0] = -tot


def _loss_call(dots2d):
    return pl.pallas_call(
        _loss_body,
        out_shape=jax.ShapeDtypeStruct((1, 1), jnp.float32),
        out_specs=pl.BlockSpec(memory_space=pltpu.SMEM),
    )(dots2d)


def kernel(center_word, outside_word, negtive_word, W_center, W_outside):
    B = center_word.shape[0]
    oc_idx = jnp.concatenate(
        [outside_word[:, None], negtive_word], axis=1).reshape(-1)
    dots = _sc_dots(center_word, oc_idx, W_center, W_outside)
    out = _loss_call(dots.reshape(B * R // 128, 128))
    return out[0, 0]


# speedup vs baseline: 1.2828x; 1.2828x over previous
import jax
import jax.numpy as jnp
from jax import lax
from jax.experimental import pallas as pl
from jax.experimental.pallas import tpu as pltpu
from jax.experimental.pallas import tpu_sc as plsc

D = 64
L = 16
NJ = D // L
NEG = 20
R = NEG + 1
BC = 32
DU = 4


def _sc_body(cidx, ocidx, wc, wo, dots_out,
             idx_c, idx_oc, c_all, oc_rows, dots_v,
             sem_c, sem_g, sem_i, sem_o):
    info = plsc.get_sparse_core_info()
    nw = info.num_cores * info.num_subcores
    B = cidx.shape[0]
    b_per_w = B // nw
    chunks = b_per_w // BC

    wid = lax.axis_index("s") * info.num_cores + lax.axis_index("c")
    w0 = pl.multiple_of(wid * b_per_w, b_per_w)

    pltpu.sync_copy(cidx.at[pl.ds(w0, b_per_w)], idx_c)
    pltpu.async_copy(wc.at[idx_c], c_all, sem_c).wait()

    def idx_cp(ch, p):
        return pltpu.make_async_copy(
            ocidx.at[pl.ds((w0 + ch * BC) * R, BC * R)],
            idx_oc.at[p], sem_i)

    def rows_cp(p):
        return pltpu.make_async_copy(
            wo.at[idx_oc.at[p]], oc_rows.at[p], sem_g.at[p])

    def dots_cp(ch, p):
        return pltpu.make_async_copy(
            dots_v.at[p],
            dots_out.at[pl.ds((w0 + ch * BC) * R, BC * R)], sem_o)

    def compute(ch, p):
        lanes = lax.iota(jnp.int32, L)
        for bb in range(BC // L):
            b_ids = ch * BC + bb * L + lanes
            ids_r = (bb * L + lanes) * R
            for (k0, k1) in ((0, 11), (11, R)):
                nk = k1 - k0
                row_ids = [ids_r + (k0 + t) for t in range(nk)]

                def dbody(d0, accs, row_ids=row_ids, b_ids=b_ids, p=p):
                    for u in range(DU):
                        d = d0 * DU + u
                        dcol = (lanes + d) & (D - 1)
                        cvec = plsc.load_gather(c_all, [b_ids, dcol])
                        accs = tuple(
                            acc + cvec * plsc.load_gather(
                                oc_rows.at[p], [row_ids[t], dcol])
                            for t, acc in enumerate(accs))
                    return accs

                accs = lax.fori_loop(
                    0, D // DU, dbody,
                    tuple(jnp.zeros((L,), jnp.float32) for _ in range(nk)))
                for t in range(nk):
                    dots_v[p, pl.ds((k0 + t) * BC + bb * L, L)] = accs[t]

    idx_cp(0, 0).start()
    idx_cp(0, 0).wait()
    rows_cp(0).start()
    idx_cp(1, 1).start()

    def chbody(ch, carry):
        p = ch & 1

        @pl.when(ch + 1 < chunks)
        def _():
            idx_cp(ch + 1, 1 - p).wait()
            rows_cp(1 - p).start()

        @pl.when(ch + 2 < chunks)
        def _():
            idx_cp(ch + 2, p).start()

        rows_cp(p).wait()

        @pl.when(ch >= 1)
        def _():
            dots_cp(ch - 1, 1 - p).wait()

        compute(ch, p)
        dots_cp(ch, p).start()
        return carry

    lax.fori_loop(0, chunks, chbody, 0)
    dots_cp(chunks - 1, (chunks - 1) & 1).wait()


def _sc_dots(center_word, oc_idx, W_center, W_outside):
    B = center_word.shape[0]
    info = plsc.get_sparse_core_info()
    nw = info.num_cores * info.num_subcores
    b_per_w = B // nw
    mesh = plsc.VectorSubcoreMesh(core_axis_name="c", subcore_axis_name="s")
    f = pl.kernel(
        _sc_body, mesh=mesh,
        compiler_params=pltpu.CompilerParams(
            needs_layout_passes=False, use_tc_tiling_on_sc=False),
        out_type=jax.ShapeDtypeStruct((B * R,), jnp.float32),
        scratch_types=[
            pltpu.VMEM((b_per_w,), jnp.int32),
            pltpu.VMEM((2, BC * R), jnp.int32),
            pltpu.VMEM((b_per_w, D), jnp.float32),
            pltpu.VMEM((2, BC * R, D), jnp.float32),
            pltpu.VMEM((2, BC * R), jnp.float32),
            pltpu.SemaphoreType.DMA,
            pltpu.SemaphoreType.DMA((2,)),
            pltpu.SemaphoreType.DMA,
            pltpu.SemaphoreType.DMA,
        ],
    )
    return f(center_word, oc_idx, W_center, W_outside)


def _loss_body(dots_ref, out_ref):
    tot = jnp.sum(jax.nn.log_sigmoid(dots_ref[...]))
    out_ref[0, 0] = -tot


def _loss_call(dots2d):
    return pl.pallas_call(
        _loss_body,
        out_shape=jax.ShapeDtypeStruct((1, 1), jnp.float32),
        out_specs=pl.BlockSpec(memory_space=pltpu.SMEM),
    )(dots2d)


def kernel(center_word, outside_word, negtive_word, W_center, W_outside):
    B = center_word.shape[0]
    oc_idx = jnp.concatenate(
        [outside_word[:, None], negtive_word], axis=1).reshape(-1)
    dots = _sc_dots(center_word, oc_idx, W_center, W_outside)
    out = _loss_call(dots.reshape(B * R // 128, 128))
    return out[0, 0]


# speedup vs baseline: 1.3430x; 1.0469x over previous
import jax
import jax.numpy as jnp
from jax import lax
from jax.experimental import pallas as pl
from jax.experimental.pallas import tpu as pltpu
from jax.experimental.pallas import tpu_sc as plsc

D = 64
DP = 128
L = 16
NEG = 20
R = NEG + 1
BC = 16
DU = 4


def _sc_body(cidx, ocidx, wc, wo, dots_out,
             idx_c, idx_oc, c_rows, oc_rows, dots_v,
             sem_c, sem_g, sem_i, sem_o):
    info = plsc.get_sparse_core_info()
    nw = info.num_cores * info.num_subcores
    B = cidx.shape[0]
    b_per_w = B // nw
    chunks = b_per_w // BC

    wid = lax.axis_index("s") * info.num_cores + lax.axis_index("c")
    w0 = pl.multiple_of(wid * b_per_w, b_per_w)

    pltpu.sync_copy(cidx.at[pl.ds(w0, b_per_w)], idx_c)

    def idx_cp(ch, p):
        return pltpu.make_async_copy(
            ocidx.at[pl.ds((w0 + ch * BC) * R, BC * R)],
            idx_oc.at[p], sem_i)

    def rows_cp(p):
        return pltpu.make_async_copy(
            wo.at[idx_oc.at[p]], oc_rows.at[p], sem_g.at[p])

    def c_cp(ch, p):
        return pltpu.make_async_copy(
            wc.at[idx_c.at[pl.ds(ch * BC, BC)]], c_rows.at[p], sem_c.at[p])

    def dots_cp(ch, p):
        return pltpu.make_async_copy(
            dots_v.at[p],
            dots_out.at[pl.ds((w0 + ch * BC) * R, BC * R)], sem_o)

    lanes = lax.iota(jnp.int32, L)

    def compute(p):
        ids_r = lanes * R
        for (k0, k1) in ((0, 11), (11, R)):
            nk = k1 - k0
            row_ids = [ids_r + (k0 + t) for t in range(nk)]

            def dbody(d0, accs, row_ids=row_ids, p=p):
                for u in range(DU):
                    d = d0 * DU + u
                    dcol = (lanes + d) & (D - 1)
                    cvec = plsc.load_gather(c_rows.at[p], [lanes, dcol])
                    accs = tuple(
                        acc + cvec * plsc.load_gather(
                            oc_rows.at[p], [row_ids[t], dcol])
                        for t, acc in enumerate(accs))
                return accs

            accs = lax.fori_loop(
                0, D // DU, dbody,
                tuple(jnp.zeros((L,), jnp.float32) for _ in range(nk)))
            for t in range(nk):
                dots_v[p, pl.ds((k0 + t) * BC, L)] = accs[t]

    idx_cp(0, 0).start()
    idx_cp(0, 0).wait()
    rows_cp(0).start()
    c_cp(0, 0).start()
    idx_cp(1, 1).start()

    def chbody(ch, carry):
        p = ch & 1

        @pl.when(ch + 1 < chunks)
        def _():
            idx_cp(ch + 1, 1 - p).wait()
            rows_cp(1 - p).start()
            c_cp(ch + 1, 1 - p).start()

        @pl.when(ch + 2 < chunks)
        def _():
            idx_cp(ch + 2, p).start()

        rows_cp(p).wait()
        c_cp(ch, p).wait()

        @pl.when(ch >= 1)
        def _():
            dots_cp(ch - 1, 1 - p).wait()

        compute(p)
        dots_cp(ch, p).start()
        return carry

    lax.fori_loop(0, chunks, chbody, 0)
    dots_cp(chunks - 1, (chunks - 1) & 1).wait()


def _sc_dots(center_word, oc_idx, W_center, W_outside):
    B = center_word.shape[0]
    info = plsc.get_sparse_core_info()
    nw = info.num_cores * info.num_subcores
    b_per_w = B // nw
    mesh = plsc.VectorSubcoreMesh(core_axis_name="c", subcore_axis_name="s")
    f = pl.kernel(
        _sc_body, mesh=mesh,
        compiler_params=pltpu.CompilerParams(
            needs_layout_passes=False, use_tc_tiling_on_sc=False),
        out_type=jax.ShapeDtypeStruct((B * R,), jnp.float32),
        scratch_types=[
            pltpu.VMEM((b_per_w,), jnp.int32),
            pltpu.VMEM((2, BC * R), jnp.int32),
            pltpu.VMEM((2, BC, DP), jnp.float32),
            pltpu.VMEM((2, BC * R, DP), jnp.float32),
            pltpu.VMEM((2, BC * R), jnp.float32),
            pltpu.SemaphoreType.DMA((2,)),
            pltpu.SemaphoreType.DMA((2,)),
            pltpu.SemaphoreType.DMA,
            pltpu.SemaphoreType.DMA,
        ],
    )
    return f(center_word, oc_idx, W_center, W_outside)


def _loss_body(dots_ref, out_ref):
    tot = jnp.sum(jax.nn.log_sigmoid(dots_ref[...]))
    out_ref[0, 0] = -tot


def _loss_call(dots2d):
    return pl.pallas_call(
        _loss_body,
        out_shape=jax.ShapeDtypeStruct((1, 1), jnp.float32),
        out_specs=pl.BlockSpec(memory_space=pltpu.SMEM),
    )(dots2d)


def kernel(center_word, outside_word, negtive_word, W_center, W_outside):
    B = center_word.shape[0]
    oc_idx = jnp.concatenate(
        [outside_word[:, None], negtive_word], axis=1).reshape(-1)
    wc_p = jnp.pad(W_center, ((0, 0), (0, DP - D)))
    wo_p = jnp.pad(W_outside, ((0, 0), (0, DP - D)))
    dots = _sc_dots(center_word, oc_idx, wc_p, wo_p)
    out = _loss_call(dots.reshape(B * R // 128, 128))
    return out[0, 0]


# speedup vs baseline: 1.7560x; 1.3075x over previous
import jax
import jax.numpy as jnp
from jax import lax
from jax.experimental import pallas as pl
from jax.experimental.pallas import tpu as pltpu
from jax.experimental.pallas import tpu_sc as plsc

D = 64
DP = 128
L = 16
NEG = 20
R = NEG + 1
BC = 16
DU = 4


def _sc_body(crows_hbm, ocidx, wo, dots_out,
             idx_oc, c_rows, oc_rows, dots_v,
             sem_c, sem_g, sem_i, sem_o):
    info = plsc.get_sparse_core_info()
    nw = info.num_cores * info.num_subcores
    B = crows_hbm.shape[0]
    b_per_w = B // nw
    chunks = b_per_w // BC

    wid = lax.axis_index("s") * info.num_cores + lax.axis_index("c")
    w0 = pl.multiple_of(wid * b_per_w, b_per_w)

    def idx_cp(ch, p):
        return pltpu.make_async_copy(
            ocidx.at[pl.ds((w0 + ch * BC) * R, BC * R)],
            idx_oc.at[p], sem_i)

    def rows_cp(p):
        return pltpu.make_async_copy(
            wo.at[idx_oc.at[p]], oc_rows.at[p], sem_g.at[p])

    def c_cp(ch, p):
        return pltpu.make_async_copy(
            crows_hbm.at[pl.ds(w0 + ch * BC, BC)], c_rows.at[p], sem_c.at[p])

    def dots_cp(ch, p):
        return pltpu.make_async_copy(
            dots_v.at[p],
            dots_out.at[pl.ds((w0 + ch * BC) * R, BC * R)], sem_o)

    lanes = lax.iota(jnp.int32, L)

    def compute(p):
        ids_r = lanes * R
        for (k0, k1) in ((0, 11), (11, R)):
            nk = k1 - k0
            row_ids = [ids_r + (k0 + t) for t in range(nk)]

            def dbody(d0, accs, row_ids=row_ids, p=p):
                for u in range(DU):
                    d = d0 * DU + u
                    dcol = (lanes + d) & (D - 1)
                    cvec = plsc.load_gather(c_rows.at[p], [lanes, dcol])
                    accs = tuple(
                        acc + cvec * plsc.load_gather(
                            oc_rows.at[p], [row_ids[t], dcol])
                        for t, acc in enumerate(accs))
                return accs

            accs = lax.fori_loop(
                0, D // DU, dbody,
                tuple(jnp.zeros((L,), jnp.float32) for _ in range(nk)))
            for t in range(nk):
                dots_v[p, pl.ds((k0 + t) * BC, L)] = accs[t]

    idx_cp(0, 0).start()
    idx_cp(0, 0).wait()
    rows_cp(0).start()
    c_cp(0, 0).start()
    idx_cp(1, 1).start()

    def chbody(ch, carry):
        p = ch & 1

        @pl.when(ch + 1 < chunks)
        def _():
            idx_cp(ch + 1, 1 - p).wait()
            rows_cp(1 - p).start()
            c_cp(ch + 1, 1 - p).start()

        @pl.when(ch + 2 < chunks)
        def _():
            idx_cp(ch + 2, p).start()

        rows_cp(p).wait()
        c_cp(ch, p).wait()

        @pl.when(ch >= 1)
        def _():
            dots_cp(ch - 1, 1 - p).wait()

        compute(p)
        dots_cp(ch, p).start()
        return carry

    lax.fori_loop(0, chunks, chbody, 0)
    dots_cp(chunks - 1, (chunks - 1) & 1).wait()


def _sc_dots(c_rows, oc_idx, W_outside):
    B = c_rows.shape[0]
    mesh = plsc.VectorSubcoreMesh(core_axis_name="c", subcore_axis_name="s")
    f = pl.kernel(
        _sc_body, mesh=mesh,
        compiler_params=pltpu.CompilerParams(
            needs_layout_passes=False, use_tc_tiling_on_sc=False),
        out_type=jax.ShapeDtypeStruct((B * R,), jnp.float32),
        scratch_types=[
            pltpu.VMEM((2, BC * R), jnp.int32),
            pltpu.VMEM((2, BC, DP), jnp.float32),
            pltpu.VMEM((2, BC * R, DP), jnp.float32),
            pltpu.VMEM((2, BC * R), jnp.float32),
            pltpu.SemaphoreType.DMA((2,)),
            pltpu.SemaphoreType.DMA((2,)),
            pltpu.SemaphoreType.DMA,
            pltpu.SemaphoreType.DMA,
        ],
    )
    return f(c_rows, oc_idx, W_outside)


def _loss_body(dots_ref, out_ref):
    tot = jnp.sum(jax.nn.log_sigmoid(dots_ref[...]))
    out_ref[0, 0] = -tot


def _loss_call(dots2d):
    return pl.pallas_call(
        _loss_body,
        out_shape=jax.ShapeDtypeStruct((1, 1), jnp.float32),
        out_specs=pl.BlockSpec(memory_space=pltpu.SMEM),
    )(dots2d)


def kernel(center_word, outside_word, negtive_word, W_center, W_outside):
    B = center_word.shape[0]
    oc_idx = jnp.concatenate(
        [outside_word[:, None], negtive_word], axis=1).reshape(-1)
    c_rows = jnp.pad(W_center[center_word], ((0, 0), (0, DP - D)))
    wo_p = jnp.pad(W_outside, ((0, 0), (0, DP - D)))
    dots = _sc_dots(c_rows, oc_idx, wo_p)
    out = _loss_call(dots.reshape(B * R // 128, 128))
    return out[0, 0]
